# baseline (speedup 1.0000x reference)
import functools

import jax
import jax.numpy as jnp
from jax import lax
from jax.experimental import pallas as pl
from jax.experimental.pallas import tpu as pltpu
from jax.experimental.pallas import tpu_sc as plsc

N = 1_000_000
NP = 1 << 20
ROWS, COLS = 1024, 1024
BLK_ROWS = 128
NBLK = ROWS // BLK_ROWS
H = 16
NEG = -1e30
STEPS = 16

NWORK = 32
SHARD = NP // NWORK
NGRP = 16
GRP = SHARD // NGRP
GVEC = GRP // 16
UNROLL = 4
GPB = BLK_ROWS * COLS // GRP
BIG = 0x7FFFFFFF


def _bf(x):
    return x.astype(jnp.bfloat16).astype(jnp.float32)


CH = 8
NCH = BLK_ROWS // CH


def _score_body(a_ref, b_ref, v_ref, x_ref, y_ref, s_ref, stat_ref, gmax_ref):
    bi = pl.program_id(0)
    r0 = jax.lax.broadcasted_iota(jnp.int32, (CH, COLS), 0)
    c0 = jax.lax.broadcasted_iota(jnp.int32, (CH, COLS), 1)
    m_run = jnp.float32(-3.0e38)
    se_run = jnp.float32(0.0)
    for ch in range(NCH):
        x = _bf(x_ref[0, ch * CH:(ch + 1) * CH, :])
        y = _bf(y_ref[0, ch * CH:(ch + 1) * CH, :])
        s = jnp.zeros_like(x)
        for j in range(H):
            z = a_ref[j] * x + b_ref[j] * y
            s = s + v_ref[j] * _bf(jnp.tanh(z))
        gidx = (bi * BLK_ROWS + ch * CH + r0) * COLS + c0
        s = jnp.where(gidx < N, s, NEG)
        s_ref[ch * CH:(ch + 1) * CH, :] = s
        ng = CH * COLS // GRP
        m4 = jnp.max(s.reshape(ng, GRP // COLS, COLS), axis=(1, 2))
        gmax_ref[0, 0, ch * ng:(ch + 1) * ng] = m4
        mc = jnp.max(s)
        m_new = jnp.maximum(m_run, mc)
        se_run = se_run * jnp.exp(m_run - m_new) + jnp.sum(jnp.exp(s - m_new))
        m_run = m_new
    lane = jax.lax.broadcasted_iota(jnp.int32, (1, 1, 128), 2)
    stat_ref[...] = jnp.where(lane == 0, m_run, jnp.where(lane == 1, se_run, 0.0))


def _scores(static, W, v):
    xy = jnp.concatenate(
        [static.reshape(2, N), jnp.zeros((2, NP - N), jnp.float32)], axis=1
    ).reshape(2, ROWS, COLS)
    a = jax.lax.reduce_precision(W[:, 0], exponent_bits=8, mantissa_bits=7)
    b = jax.lax.reduce_precision(W[:, 1], exponent_bits=8, mantissa_bits=7)
    v = jax.lax.reduce_precision(v, exponent_bits=8, mantissa_bits=7)
    smem_spec = pl.BlockSpec(memory_space=pltpu.SMEM)
    s0, stats, gmax = pl.pallas_call(
        _score_body,
        grid=(NBLK,),
        in_specs=[
            smem_spec,
            smem_spec,
            smem_spec,
            pl.BlockSpec((1, BLK_ROWS, COLS), lambda i: (0, i, 0)),
            pl.BlockSpec((1, BLK_ROWS, COLS), lambda i: (1, i, 0)),
        ],
        out_specs=[
            pl.BlockSpec((BLK_ROWS, COLS), lambda i: (i, 0)),
            pl.BlockSpec((1, 1, 128), lambda i: (i, 0, 0)),
            pl.BlockSpec((1, 1, GPB), lambda i: (i, 0, 0)),
        ],
        out_shape=[
            jax.ShapeDtypeStruct((ROWS, COLS), jnp.float32),
            jax.ShapeDtypeStruct((NBLK, 1, 128), jnp.float32),
            jax.ShapeDtypeStruct((NBLK, 1, GPB), jnp.float32),
        ],
    )(a, b, v, xy, xy)
    return s0, stats, gmax


def _lperm(x, idx):
    dn = lax.GatherDimensionNumbers(
        offset_dims=(), collapsed_slice_dims=(0,), start_index_map=(0,))
    return lax.gather(x, idx.reshape(16, 1), dn, slice_sizes=(1,),
                      mode=lax.GatherScatterMode.PROMISE_IN_BOUNDS)


def _allmax(x, lane):
    for s in (1, 2, 4, 8):
        x = jnp.maximum(x, _lperm(x, lane ^ s))
    return x


def _allmin(x, lane):
    for s in (1, 2, 4, 8):
        x = jnp.minimum(x, _lperm(x, lane ^ s))
    return x


def _sc_topk_body(s0_hbm, gmax_hbm, vals_hbm, idx_hbm, data_v, sg_v, vrow_v,
                  irow_v):
    wid = lax.axis_index("s") * 2 + lax.axis_index("c")
    pltpu.sync_copy(s0_hbm.at[pl.ds(wid * SHARD, SHARD)], data_v)
    pltpu.sync_copy(gmax_hbm.at[pl.ds(wid * NGRP, NGRP)], sg_v)

    lane = lax.iota(jnp.int32, 16)
    fneg = jnp.full((16,), -3.0e38, jnp.float32)
    sg = sg_v[...]

    def group_scan(g_base):
        def body(i, carry):
            v1, v2, i1 = carry
            for u in range(UNROLL):
                base = g_base + (i * UNROLL + u) * 16
                x = data_v[pl.ds(base, 16)]
                idxv = base + lane
                upd = x > v1
                v2 = jnp.where(upd, v1, jnp.where(x > v2, x, v2))
                i1 = jnp.where(upd, idxv, i1)
                v1 = jnp.where(upd, x, v1)
            return v1, v2, i1

        return lax.fori_loop(
            0, GVEC // UNROLL, body,
            (fneg, fneg, jnp.full((16,), BIG, jnp.int32)))

    vrow = fneg
    irow = jnp.zeros((16,), jnp.int32)
    for t in range(STEPS):
        mtv = _allmax(sg, lane)
        gsv = _allmin(jnp.where(sg == mtv, lane, BIG), lane)
        v1, v2, i1 = group_scan(gsv[0] * GRP)
        mgv = _allmax(v1, lane)
        posv = _allmin(jnp.where(v1 == mgv, i1, BIG), lane)
        pos = posv[0]
        wl = pos & 15
        wbase = pos - wl
        w = data_v[pl.ds(wbase, 16)]
        data_v[pl.ds(wbase, 16)] = jnp.where(lane == wl, NEG, w)
        sg_new = _allmax(jnp.where(lane == wl, v2, v1), lane)
        sg = jnp.where(lane == gsv, sg_new, sg)
        vrow = jnp.where(lane == t, mgv, vrow)
        irow = jnp.where(lane == t, posv + wid * SHARD, irow)

    vrow_v[...] = vrow
    irow_v[...] = irow
    pltpu.sync_copy(vrow_v, vals_hbm.at[wid])
    pltpu.sync_copy(irow_v, idx_hbm.at[wid])


def _sc_topk(s0_flat, gmax_flat):
    mesh = plsc.VectorSubcoreMesh(core_axis_name="c", subcore_axis_name="s")
    kfn = functools.partial(
        pl.kernel,
        mesh=mesh,
        out_type=[
            jax.ShapeDtypeStruct((NWORK, 16), jnp.float32),
            jax.ShapeDtypeStruct((NWORK, 16), jnp.int32),
        ],
        scratch_types=[
            pltpu.VMEM((SHARD,), jnp.float32),
            pltpu.VMEM((16,), jnp.float32),
            pltpu.VMEM((16,), jnp.float32),
            pltpu.VMEM((16,), jnp.int32),
        ],
    )(_sc_topk_body)
    return kfn(s0_flat, gmax_flat)


def _merge_body(v_ref, i_ref, st_ref, ti_ref, lp_ref):
    V = v_ref[...]
    I = i_ref[...]
    st = st_ref[...]
    i2 = jax.lax.broadcasted_iota(jnp.int32, (NBLK, 1, 128), 2)
    mcol = jnp.max(jnp.where(i2 == 0, st, NEG), axis=2, keepdims=True)
    scol = jnp.sum(jnp.where(i2 == 1, st, 0.0), axis=2, keepdims=True)
    M = jnp.max(mcol)
    S = jnp.sum(jnp.exp(mcol - M) * scol)
    lane16 = jax.lax.broadcasted_iota(jnp.int32, (1, 16), 1)
    ti = jnp.zeros((1, 16), jnp.int32)
    lp = jnp.zeros((1, 16), jnp.float32)
    for t in range(STEPS):
        cur = jnp.max(V)
        pick = jnp.min(jnp.where(V == cur, I, BIG))
        V = jnp.where((V == cur) & (I == pick), NEG, V)
        lp = jnp.where(lane16 == t, cur - M - jnp.log(S), lp)
        ti = jnp.where(lane16 == t, pick, ti)
        S = S - jnp.exp(cur - M)
    ti_ref[...] = ti
    lp_ref[...] = lp


def _merge(cand_v, cand_i, stats):
    return pl.pallas_call(
        _merge_body,
        out_shape=[
            jax.ShapeDtypeStruct((1, STEPS), jnp.int32),
            jax.ShapeDtypeStruct((1, STEPS), jnp.float32),
        ],
    )(cand_v, cand_i, stats)


def kernel(static, dynamic, station_num_lim, W, v):
    s0, stats, gmax = _scores(static, W, v)
    cand_v, cand_i = _sc_topk(s0.reshape(-1), gmax.reshape(-1))
    tour_idx, tour_logp = _merge(cand_v, cand_i, stats)
    return tour_idx, tour_logp


# speedup vs baseline: 1.0539x; 1.0539x over previous
import functools

import jax
import jax.numpy as jnp
from jax import lax
from jax.experimental import pallas as pl
from jax.experimental.pallas import tpu as pltpu
from jax.experimental.pallas import tpu_sc as plsc

N = 1_000_000
NP = 1 << 20
ROWS, COLS = 1024, 1024
BLK_ROWS = 128
NBLK = ROWS // BLK_ROWS
H = 16
NEG = -1e30
STEPS = 16

NWORK = 32
SHARD = NP // NWORK
NGRP = 32
GRP = SHARD // NGRP
GVEC = GRP // 16
UNROLL = 4
GPB = BLK_ROWS * COLS // GRP
BIG = 0x7FFFFFFF


def _bf(x):
    return x.astype(jnp.bfloat16).astype(jnp.float32)


CH = 8
NCH = BLK_ROWS // CH


def _score_body(a_ref, b_ref, v_ref, x_ref, y_ref, s_ref, stat_ref, gmax_ref):
    bi = pl.program_id(0)
    r0 = jax.lax.broadcasted_iota(jnp.int32, (CH, COLS), 0)
    c0 = jax.lax.broadcasted_iota(jnp.int32, (CH, COLS), 1)
    m_run = jnp.float32(-3.0e38)
    se_run = jnp.float32(0.0)
    for ch in range(NCH):
        x = _bf(x_ref[0, ch * CH:(ch + 1) * CH, :])
        y = _bf(y_ref[0, ch * CH:(ch + 1) * CH, :])
        s = jnp.zeros_like(x)
        for j in range(H):
            z = a_ref[j] * x + b_ref[j] * y
            s = s + v_ref[j] * _bf(jnp.tanh(z))
        gidx = (bi * BLK_ROWS + ch * CH + r0) * COLS + c0
        s = jnp.where(gidx < N, s, NEG)
        s_ref[ch * CH:(ch + 1) * CH, :] = s
        gmax_ref[0, 0, ch * CH:(ch + 1) * CH] = jnp.max(s, axis=1)
        mc = jnp.max(s)
        m_new = jnp.maximum(m_run, mc)
        se_run = se_run * jnp.exp(m_run - m_new) + jnp.sum(jnp.exp(s - m_new))
        m_run = m_new
    lane = jax.lax.broadcasted_iota(jnp.int32, (1, 1, 128), 2)
    stat_ref[...] = jnp.where(lane == 0, m_run, jnp.where(lane == 1, se_run, 0.0))


def _scores(static, W, v):
    xy = jnp.concatenate(
        [static.reshape(2, N), jnp.zeros((2, NP - N), jnp.float32)], axis=1
    ).reshape(2, ROWS, COLS)
    a = jax.lax.reduce_precision(W[:, 0], exponent_bits=8, mantissa_bits=7)
    b = jax.lax.reduce_precision(W[:, 1], exponent_bits=8, mantissa_bits=7)
    v = jax.lax.reduce_precision(v, exponent_bits=8, mantissa_bits=7)
    smem_spec = pl.BlockSpec(memory_space=pltpu.SMEM)
    s0, stats, gmax = pl.pallas_call(
        _score_body,
        grid=(NBLK,),
        in_specs=[
            smem_spec,
            smem_spec,
            smem_spec,
            pl.BlockSpec((1, BLK_ROWS, COLS), lambda i: (0, i, 0)),
            pl.BlockSpec((1, BLK_ROWS, COLS), lambda i: (1, i, 0)),
        ],
        out_specs=[
            pl.BlockSpec((BLK_ROWS, COLS), lambda i: (i, 0)),
            pl.BlockSpec((1, 1, 128), lambda i: (i, 0, 0)),
            pl.BlockSpec((1, 1, GPB), lambda i: (i, 0, 0)),
        ],
        out_shape=[
            jax.ShapeDtypeStruct((ROWS, COLS), jnp.float32),
            jax.ShapeDtypeStruct((NBLK, 1, 128), jnp.float32),
            jax.ShapeDtypeStruct((NBLK, 1, GPB), jnp.float32),
        ],
    )(a, b, v, xy, xy)
    return s0, stats, gmax


def _lperm(x, idx):
    dn = lax.GatherDimensionNumbers(
        offset_dims=(), collapsed_slice_dims=(0,), start_index_map=(0,))
    return lax.gather(x, idx.reshape(16, 1), dn, slice_sizes=(1,),
                      mode=lax.GatherScatterMode.PROMISE_IN_BOUNDS)


def _allmax(x, lane):
    for s in (1, 2, 4, 8):
        x = jnp.maximum(x, _lperm(x, lane ^ s))
    return x


def _allmin(x, lane):
    for s in (1, 2, 4, 8):
        x = jnp.minimum(x, _lperm(x, lane ^ s))
    return x


def _sc_topk_body(s0_hbm, gmax_hbm, vals_hbm, idx_hbm, data_v, sg_v, vrow_v,
                  irow_v):
    wid = lax.axis_index("s") * 2 + lax.axis_index("c")
    pltpu.sync_copy(s0_hbm.at[pl.ds(wid * SHARD, SHARD)], data_v)
    pltpu.sync_copy(gmax_hbm.at[pl.ds(wid * NGRP, NGRP)], sg_v)

    lane = lax.iota(jnp.int32, 16)
    fneg = jnp.full((16,), -3.0e38, jnp.float32)
    sg0 = sg_v[pl.ds(0, 16)]
    sg1 = sg_v[pl.ds(16, 16)]

    def group_scan(g_base):
        def body(i, carry):
            v1, v2, i1 = carry
            for u in range(UNROLL):
                base = g_base + (i * UNROLL + u) * 16
                x = data_v[pl.ds(base, 16)]
                idxv = base + lane
                upd = x > v1
                v2 = jnp.where(upd, v1, jnp.where(x > v2, x, v2))
                i1 = jnp.where(upd, idxv, i1)
                v1 = jnp.where(upd, x, v1)
            return v1, v2, i1

        return lax.fori_loop(
            0, GVEC // UNROLL, body,
            (fneg, fneg, jnp.full((16,), BIG, jnp.int32)))

    vrow = fneg
    irow = jnp.zeros((16,), jnp.int32)
    for t in range(STEPS):
        mtv = _allmax(jnp.maximum(sg0, sg1), lane)
        idc0 = jnp.where(sg0 == mtv, lane, BIG)
        idc1 = jnp.where(sg1 == mtv, lane + 16, BIG)
        gsv = _allmin(jnp.minimum(idc0, idc1), lane)
        v1, v2, i1 = group_scan(gsv[0] * GRP)
        mgv = _allmax(v1, lane)
        posv = _allmin(jnp.where(v1 == mgv, i1, BIG), lane)
        pos = posv[0]
        wl = pos & 15
        wbase = pos - wl
        w = data_v[pl.ds(wbase, 16)]
        data_v[pl.ds(wbase, 16)] = jnp.where(lane == wl, NEG, w)
        sg_new = _allmax(jnp.where(lane == wl, v2, v1), lane)
        sg0 = jnp.where(lane == gsv, sg_new, sg0)
        sg1 = jnp.where(lane + 16 == gsv, sg_new, sg1)
        vrow = jnp.where(lane == t, mgv, vrow)
        irow = jnp.where(lane == t, posv + wid * SHARD, irow)

    vrow_v[...] = vrow
    irow_v[...] = irow
    pltpu.sync_copy(vrow_v, vals_hbm.at[wid])
    pltpu.sync_copy(irow_v, idx_hbm.at[wid])


def _sc_topk(s0_flat, gmax_flat):
    mesh = plsc.VectorSubcoreMesh(core_axis_name="c", subcore_axis_name="s")
    kfn = functools.partial(
        pl.kernel,
        mesh=mesh,
        out_type=[
            jax.ShapeDtypeStruct((NWORK, 16), jnp.float32),
            jax.ShapeDtypeStruct((NWORK, 16), jnp.int32),
        ],
        scratch_types=[
            pltpu.VMEM((SHARD,), jnp.float32),
            pltpu.VMEM((NGRP,), jnp.float32),
            pltpu.VMEM((16,), jnp.float32),
            pltpu.VMEM((16,), jnp.int32),
        ],
    )(_sc_topk_body)
    return kfn(s0_flat, gmax_flat)


def _merge_body(v_ref, i_ref, st_ref, ti_ref, lp_ref):
    V = v_ref[...]
    I = i_ref[...]
    st = st_ref[...]
    i2 = jax.lax.broadcasted_iota(jnp.int32, (NBLK, 1, 128), 2)
    mcol = jnp.max(jnp.where(i2 == 0, st, NEG), axis=2, keepdims=True)
    scol = jnp.sum(jnp.where(i2 == 1, st, 0.0), axis=2, keepdims=True)
    M = jnp.max(mcol)
    S = jnp.sum(jnp.exp(mcol - M) * scol)
    lane16 = jax.lax.broadcasted_iota(jnp.int32, (1, 16), 1)
    ti = jnp.zeros((1, 16), jnp.int32)
    lp = jnp.zeros((1, 16), jnp.float32)
    for t in range(STEPS):
        cur = jnp.max(V)
        pick = jnp.min(jnp.where(V == cur, I, BIG))
        V = jnp.where((V == cur) & (I == pick), NEG, V)
        lp = jnp.where(lane16 == t, cur - M - jnp.log(S), lp)
        ti = jnp.where(lane16 == t, pick, ti)
        S = S - jnp.exp(cur - M)
    ti_ref[...] = ti
    lp_ref[...] = lp


def _merge(cand_v, cand_i, stats):
    return pl.pallas_call(
        _merge_body,
        out_shape=[
            jax.ShapeDtypeStruct((1, STEPS), jnp.int32),
            jax.ShapeDtypeStruct((1, STEPS), jnp.float32),
        ],
    )(cand_v, cand_i, stats)


def kernel(static, dynamic, station_num_lim, W, v):
    s0, stats, gmax = _scores(static, W, v)
    cand_v, cand_i = _sc_topk(s0.reshape(-1), gmax.reshape(-1))
    tour_idx, tour_logp = _merge(cand_v, cand_i, stats)
    return tour_idx, tour_logp


# speedup vs baseline: 1.1218x; 1.0644x over previous
import functools

import jax
import jax.numpy as jnp
from jax import lax
from jax.experimental import pallas as pl
from jax.experimental.pallas import tpu as pltpu
from jax.experimental.pallas import tpu_sc as plsc

N = 1_000_000
NP = 1 << 20
ROWS, COLS = 1024, 1024
BLK_ROWS = 128
NBLK = ROWS // BLK_ROWS
H = 16
NEG = -1e30
STEPS = 16

NWORK = 32
SHARD = NP // NWORK
NGRP = 32
GRP = SHARD // NGRP
GVEC = GRP // 16
UNROLL = 4
GPB = BLK_ROWS * COLS // GRP
BIG = 0x7FFFFFFF


def _bf(x):
    return x.astype(jnp.bfloat16).astype(jnp.float32)


CH = 8
NCH = BLK_ROWS // CH


def _score_body(a_ref, b_ref, v_ref, x_ref, y_ref, s_ref, stat_ref, gmax_ref):
    bi = pl.program_id(0)
    r0 = jax.lax.broadcasted_iota(jnp.int32, (CH, COLS), 0)
    c0 = jax.lax.broadcasted_iota(jnp.int32, (CH, COLS), 1)
    m_run = jnp.float32(-3.0e38)
    se_run = jnp.float32(0.0)
    for ch in range(NCH):
        x = _bf(x_ref[0, ch * CH:(ch + 1) * CH, :])
        y = _bf(y_ref[0, ch * CH:(ch + 1) * CH, :])
        s = jnp.zeros_like(x)
        for j in range(H):
            z = a_ref[j] * x + b_ref[j] * y
            s = s + v_ref[j] * _bf(jnp.tanh(z))
        gidx = (bi * BLK_ROWS + ch * CH + r0) * COLS + c0
        s = jnp.where(gidx < N, s, NEG)
        s_ref[ch * CH:(ch + 1) * CH, :] = s
        gmax_ref[0, 0, ch * CH:(ch + 1) * CH] = jnp.max(s, axis=1)
        mc = jnp.max(s)
        m_new = jnp.maximum(m_run, mc)
        se_run = se_run * jnp.exp(m_run - m_new) + jnp.sum(jnp.exp(s - m_new))
        m_run = m_new
    lane = jax.lax.broadcasted_iota(jnp.int32, (1, 1, 128), 2)
    stat_ref[...] = jnp.where(lane == 0, m_run, jnp.where(lane == 1, se_run, 0.0))


def _scores(static, W, v):
    xy = jnp.concatenate(
        [static.reshape(2, N), jnp.zeros((2, NP - N), jnp.float32)], axis=1
    ).reshape(2, ROWS, COLS)
    a = jax.lax.reduce_precision(W[:, 0], exponent_bits=8, mantissa_bits=7)
    b = jax.lax.reduce_precision(W[:, 1], exponent_bits=8, mantissa_bits=7)
    v = jax.lax.reduce_precision(v, exponent_bits=8, mantissa_bits=7)
    smem_spec = pl.BlockSpec(memory_space=pltpu.SMEM)
    s0, stats, gmax = pl.pallas_call(
        _score_body,
        grid=(NBLK,),
        in_specs=[
            smem_spec,
            smem_spec,
            smem_spec,
            pl.BlockSpec((1, BLK_ROWS, COLS), lambda i: (0, i, 0)),
            pl.BlockSpec((1, BLK_ROWS, COLS), lambda i: (1, i, 0)),
        ],
        out_specs=[
            pl.BlockSpec((BLK_ROWS, COLS), lambda i: (i, 0)),
            pl.BlockSpec((1, 1, 128), lambda i: (i, 0, 0)),
            pl.BlockSpec((1, 1, GPB), lambda i: (i, 0, 0)),
        ],
        out_shape=[
            jax.ShapeDtypeStruct((ROWS, COLS), jnp.float32),
            jax.ShapeDtypeStruct((NBLK, 1, 128), jnp.float32),
            jax.ShapeDtypeStruct((NBLK, 1, GPB), jnp.float32),
        ],
    )(a, b, v, xy, xy)
    return s0, stats, gmax


def _lperm(x, idx):
    dn = lax.GatherDimensionNumbers(
        offset_dims=(), collapsed_slice_dims=(0,), start_index_map=(0,))
    return lax.gather(x, idx.reshape(16, 1), dn, slice_sizes=(1,),
                      mode=lax.GatherScatterMode.PROMISE_IN_BOUNDS)


def _allmax(x, lane):
    for s in (1, 2, 4, 8):
        x = jnp.maximum(x, _lperm(x, lane ^ s))
    return x


def _allmin(x, lane):
    for s in (1, 2, 4, 8):
        x = jnp.minimum(x, _lperm(x, lane ^ s))
    return x


def _sc_topk_body(s0_hbm, gmax_hbm, vals_hbm, idx_hbm, data_v, sg_v, vrow_v,
                  irow_v):
    wid = lax.axis_index("s") * 2 + lax.axis_index("c")
    pltpu.sync_copy(s0_hbm.at[pl.ds(wid * NGRP, NGRP)], data_v)
    pltpu.sync_copy(gmax_hbm.at[pl.ds(wid * NGRP, NGRP)], sg_v)

    lane = lax.iota(jnp.int32, 16)
    fneg = jnp.full((16,), -3.0e38, jnp.float32)
    sg0 = sg_v[pl.ds(0, 16)]
    sg1 = sg_v[pl.ds(16, 16)]

    def group_scan(grow):
        def body(i, carry):
            v1, v2, i1 = carry
            for u in range(UNROLL):
                col = pl.multiple_of((i * UNROLL + u) * 16, 16)
                x = data_v[grow, pl.ds(col, 16)]
                idxv = grow * GRP + col + lane
                upd = x > v1
                v2 = jnp.where(upd, v1, jnp.where(x > v2, x, v2))
                i1 = jnp.where(upd, idxv, i1)
                v1 = jnp.where(upd, x, v1)
            return v1, v2, i1

        return lax.fori_loop(
            0, GVEC // UNROLL, body,
            (fneg, fneg, jnp.full((16,), BIG, jnp.int32)))

    vrow = fneg
    irow = jnp.zeros((16,), jnp.int32)
    for t in range(STEPS):
        mtv = _allmax(jnp.maximum(sg0, sg1), lane)
        idc0 = jnp.where(sg0 == mtv, lane, BIG)
        idc1 = jnp.where(sg1 == mtv, lane + 16, BIG)
        gsv = _allmin(jnp.minimum(idc0, idc1), lane)
        v1, v2, i1 = group_scan(gsv[0])
        mgv = _allmax(v1, lane)
        posv = _allmin(jnp.where(v1 == mgv, i1, BIG), lane)
        pos = posv[0]
        wl = pos & 15
        prow = lax.shift_right_logical(pos, 10)
        pcol = pl.multiple_of((pos - wl) & (GRP - 1), 16)
        w = data_v[prow, pl.ds(pcol, 16)]
        data_v[prow, pl.ds(pcol, 16)] = jnp.where(lane == wl, NEG, w)
        sg_new = _allmax(jnp.where(lane == wl, v2, v1), lane)
        sg0 = jnp.where(lane == gsv, sg_new, sg0)
        sg1 = jnp.where(lane + 16 == gsv, sg_new, sg1)
        vrow = jnp.where(lane == t, mgv, vrow)
        irow = jnp.where(lane == t, posv + wid * SHARD, irow)

    vrow_v[...] = vrow
    irow_v[...] = irow
    pltpu.sync_copy(vrow_v, vals_hbm.at[wid])
    pltpu.sync_copy(irow_v, idx_hbm.at[wid])


def _sc_topk(s0_flat, gmax_flat):
    mesh = plsc.VectorSubcoreMesh(core_axis_name="c", subcore_axis_name="s")
    kfn = functools.partial(
        pl.kernel,
        mesh=mesh,
        out_type=[
            jax.ShapeDtypeStruct((NWORK, 16), jnp.float32),
            jax.ShapeDtypeStruct((NWORK, 16), jnp.int32),
        ],
        scratch_types=[
            pltpu.VMEM((NGRP, GRP), jnp.float32),
            pltpu.VMEM((NGRP,), jnp.float32),
            pltpu.VMEM((16,), jnp.float32),
            pltpu.VMEM((16,), jnp.int32),
        ],
    )(_sc_topk_body)
    return kfn(s0_flat, gmax_flat)


def _merge_body(v_ref, i_ref, st_ref, ti_ref, lp_ref):
    V = v_ref[...]
    I = i_ref[...]
    st = st_ref[...]
    i2 = jax.lax.broadcasted_iota(jnp.int32, (NBLK, 1, 128), 2)
    mcol = jnp.max(jnp.where(i2 == 0, st, NEG), axis=2, keepdims=True)
    scol = jnp.sum(jnp.where(i2 == 1, st, 0.0), axis=2, keepdims=True)
    M = jnp.max(mcol)
    S = jnp.sum(jnp.exp(mcol - M) * scol)
    lane16 = jax.lax.broadcasted_iota(jnp.int32, (1, 16), 1)
    ti = jnp.zeros((1, 16), jnp.int32)
    lp = jnp.zeros((1, 16), jnp.float32)
    for t in range(STEPS):
        cur = jnp.max(V)
        pick = jnp.min(jnp.where(V == cur, I, BIG))
        V = jnp.where((V == cur) & (I == pick), NEG, V)
        lp = jnp.where(lane16 == t, cur - M - jnp.log(S), lp)
        ti = jnp.where(lane16 == t, pick, ti)
        S = S - jnp.exp(cur - M)
    ti_ref[...] = ti
    lp_ref[...] = lp


def _merge(cand_v, cand_i, stats):
    return pl.pallas_call(
        _merge_body,
        out_shape=[
            jax.ShapeDtypeStruct((1, STEPS), jnp.int32),
            jax.ShapeDtypeStruct((1, STEPS), jnp.float32),
        ],
    )(cand_v, cand_i, stats)


def kernel(static, dynamic, station_num_lim, W, v):
    s0, stats, gmax = _scores(static, W, v)
    cand_v, cand_i = _sc_topk(s0, gmax.reshape(-1))
    tour_idx, tour_logp = _merge(cand_v, cand_i, stats)
    return tour_idx, tour_logp
